# baseline (speedup 1.0000x reference)
import functools

import jax
import jax.numpy as jnp
from jax.experimental import pallas as pl

_F32 = jnp.float32
_BF16 = jnp.bfloat16
_NUM_FINE_LAYERS = 3
_NUM_COARSE_LAYERS = 6


def _edge_updates(x_i, x_j, ea, w1, b1, w2, b2, block_e):
    num_e = x_i.shape[0]

    def kern(xi_ref, xj_ref, ea_ref, w1_ref, b1_ref, w2_ref, b2_ref, o_ref):
        msg = jnp.concatenate([xi_ref[...], xj_ref[...], ea_ref[...]], axis=-1)
        u = jax.lax.dot_general(msg, w1_ref[...], (((1,), (0,)), ((), ())),
                                preferred_element_type=_F32) + b1_ref[...]
        h1 = jnp.maximum(u, 0.0).astype(_BF16)
        o_ref[...] = jax.lax.dot_general(h1, w2_ref[...], (((1,), (0,)), ((), ())),
                                         preferred_element_type=_F32) + b2_ref[...]

    return pl.pallas_call(
        kern,
        grid=(num_e // block_e,),
        in_specs=[
            pl.BlockSpec((block_e, 128), lambda i: (i, 0)),
            pl.BlockSpec((block_e, 128), lambda i: (i, 0)),
            pl.BlockSpec((block_e, 4), lambda i: (i, 0)),
            pl.BlockSpec((260, 128), lambda i: (0, 0)),
            pl.BlockSpec((1, 128), lambda i: (0, 0)),
            pl.BlockSpec((128, 128), lambda i: (0, 0)),
            pl.BlockSpec((1, 128), lambda i: (0, 0)),
        ],
        out_specs=pl.BlockSpec((block_e, 128), lambda i: (i, 0)),
        out_shape=jax.ShapeDtypeStruct((num_e, 128), _F32),
    )(x_i, x_j, ea, w1, b1, w2, b2)


def _mm_bias(x, w, b, block_m):
    m, k = x.shape
    n = w.shape[1]

    def kern(x_ref, w_ref, b_ref, o_ref):
        o_ref[...] = jax.lax.dot_general(x_ref[...], w_ref[...], (((1,), (0,)), ((), ())),
                                         preferred_element_type=_F32) + b_ref[...]

    return pl.pallas_call(
        kern,
        grid=(m // block_m,),
        in_specs=[
            pl.BlockSpec((block_m, k), lambda i: (i, 0)),
            pl.BlockSpec((k, n), lambda i: (0, 0)),
            pl.BlockSpec((1, n), lambda i: (0, 0)),
        ],
        out_specs=pl.BlockSpec((block_m, n), lambda i: (i, 0)),
        out_shape=jax.ShapeDtypeStruct((m, n), _F32),
    )(x, w, b)


def kernel(fine_x, fine_edge_index, fine_edge_attr, coarse_x, coarse_edge_index,
           coarse_edge_attr, assign, fine_enc_W, fine_enc_b, coarse_enc_W,
           coarse_enc_b, fine_W1, fine_b1, fine_W2, fine_b2, coarse_W1,
           coarse_b1, coarse_W2, coarse_b2, dec_W, dec_b):
    n_fine = fine_x.shape[0]
    n_coarse = coarse_x.shape[0]

    perm_f = jnp.argsort(fine_edge_index[1], stable=True)
    fdst = fine_edge_index[1][perm_f]
    fsrc = fine_edge_index[0][perm_f]
    fea16 = fine_edge_attr[perm_f].astype(_BF16)

    cdst = coarse_edge_index[1]
    csrc = coarse_edge_index[0]
    cea16 = coarse_edge_attr.astype(_BF16)

    def fine_mpnn(h, i):
        hb = h.astype(_BF16)
        x_i = jnp.take(hb, fdst, axis=0)
        x_j = jnp.take(hb, fsrc, axis=0)
        upd = _edge_updates(x_i, x_j, fea16,
                            fine_W1[i].astype(_BF16), fine_b1[i].reshape(1, 128),
                            fine_W2[i].astype(_BF16), fine_b2[i].reshape(1, 128),
                            block_e=2000)
        return jax.ops.segment_sum(upd, fdst, num_segments=n_fine)

    def coarse_mpnn(h, i):
        hb = h.astype(_BF16)
        x_i = jnp.take(hb, cdst, axis=0)
        x_j = jnp.take(hb, csrc, axis=0)
        upd = _edge_updates(x_i, x_j, cea16,
                            coarse_W1[i].astype(_BF16), coarse_b1[i].reshape(1, 128),
                            coarse_W2[i].astype(_BF16), coarse_b2[i].reshape(1, 128),
                            block_e=2000)
        return jax.ops.segment_sum(upd, cdst, num_segments=n_coarse)

    h_f = _mm_bias(fine_x, fine_enc_W, fine_enc_b.reshape(1, 128), block_m=400)
    h_c = _mm_bias(coarse_x, coarse_enc_W, coarse_enc_b.reshape(1, 128), block_m=200)
    h_c = h_c + assign.T @ h_f
    for i in range(_NUM_FINE_LAYERS):
        h_f = h_f + fine_mpnn(h_f, i)
        h_c = h_c + coarse_mpnn(h_c, i)
        h_f = h_f + assign @ h_c
        h_c = h_c + assign.T @ h_f
    for i in range(_NUM_FINE_LAYERS, _NUM_COARSE_LAYERS):
        h_c = h_c + coarse_mpnn(h_c, i)
        h_f = h_f + assign @ h_c
    return _mm_bias(h_f, dec_W, dec_b.reshape(1, 128), block_m=400)
